# baseline (speedup 1.0000x reference)
import functools

import jax
import numpy as np
import jax.numpy as jnp
from jax import lax
from jax.experimental import pallas as pl
from jax.experimental.pallas import tpu as pltpu
from jax.experimental.pallas import tpu_sc as plsc

_B = 32
_N = 8192
_NQ = 1024
_SUB, _LANE = 64, 128
_L = 16
_CHUNK = _N // _L


def _rotl(x, r):
    return lax.shift_left(x, np.int32(r)) | lax.shift_right_logical(
        x, np.int32(32 - r))


def _threefry_bits(k0, k1, idx):
    rot = ((13, 15, 26, 6), (17, 29, 16, 24))
    ks = (k0, k1, k0 ^ k1 ^ np.int32(0x1BD11BDA))
    x0 = jnp.full_like(idx, 0) + ks[0]
    x1 = idx + ks[1]
    for i in range(5):
        for r in rot[i % 2]:
            x0 = x0 + x1
            x1 = _rotl(x1, r) ^ x0
        x0 = x0 + ks[(i + 1) % 3]
        x1 = x1 + ks[(i + 2) % 3] + np.int32(i + 1)
    return x0 ^ x1


def _bits_kernel(sk_ref, out_ref):
    r = lax.broadcasted_iota(jnp.int32, (_SUB, _LANE), 0)
    c = lax.broadcasted_iota(jnp.int32, (_SUB, _LANE), 1)
    idx = r * _LANE + c
    out_ref[0, 0] = _threefry_bits(sk_ref[0, 0, 0], sk_ref[0, 0, 1], idx)
    out_ref[0, 1] = _threefry_bits(sk_ref[0, 0, 2], sk_ref[0, 0, 3], idx)


def _radix_pass(src_k, src_v, dst_k, dst_v, hist, shift):
    lane = lax.iota(jnp.int32, _L)
    ones = jnp.ones((_L,), jnp.int32)
    zeros = jnp.zeros((_L,), jnp.int32)
    sh = np.int32(shift)
    m255 = np.int32(255)

    def zero_body(m, carry):
        plsc.store_scatter(hist, [m * _L + lane], zeros)
        return carry
    lax.fori_loop(0, 256, zero_body, np.int32(0))

    def hist_body(t, carry):
        k = plsc.load_gather(src_k, [lane * _CHUNK + t])
        d = lax.shift_right_logical(k, sh) & m255
        plsc.addupdate_scatter(hist, [d * _L + lane], ones)
        return carry
    lax.fori_loop(0, _CHUNK, hist_body, np.int32(0))

    def scan_body(m, carry):
        h = plsc.load_gather(hist, [m * _L + lane])
        incl = plsc.cumsum(h)
        plsc.store_scatter(hist, [m * _L + lane], incl - h + carry)
        return carry + jnp.sum(h)
    lax.fori_loop(0, 256, scan_body, np.int32(0))

    def scat_body(t, carry):
        idx = lane * _CHUNK + t
        k = plsc.load_gather(src_k, [idx])
        v = plsc.load_gather(src_v, [idx])
        d = lax.shift_right_logical(k, sh) & m255
        slot = d * _L + lane
        pos = plsc.load_gather(hist, [slot])
        plsc.store_scatter(dst_k, [pos], k)
        plsc.store_scatter(dst_v, [pos], v)
        plsc.addupdate_scatter(hist, [slot], ones)
        return carry
    lax.fori_loop(0, _CHUNK, scat_body, np.int32(0))


_sc_mesh = plsc.VectorSubcoreMesh(core_axis_name="c", subcore_axis_name="s")


@functools.partial(
    pl.kernel,
    mesh=_sc_mesh,
    compiler_params=pltpu.CompilerParams(needs_layout_passes=False),
    out_type=jax.ShapeDtypeStruct((_B, _NQ), jnp.int32),
    scratch_types=[
        pltpu.VMEM((_N,), jnp.int32),
        pltpu.VMEM((_N,), jnp.int32),
        pltpu.VMEM((_N,), jnp.int32),
        pltpu.VMEM((_N,), jnp.int32),
        pltpu.VMEM((_N,), jnp.int32),
        pltpu.VMEM((256 * _L,), jnp.int32),
    ],
)
def _sc_sort(bits_hbm, out_hbm, ka, kb, va, vb, k2, hist):
    wid = lax.axis_index("s") * 2 + lax.axis_index("c")
    pltpu.sync_copy(bits_hbm.at[wid, 0], ka)
    pltpu.sync_copy(bits_hbm.at[wid, 1], k2)

    lane = lax.iota(jnp.int32, _L)

    def init_body(m, carry):
        plsc.store_scatter(va, [m * _L + lane], m * _L + lane)
        return carry
    lax.fori_loop(0, _N // _L, init_body, np.int32(0))

    _radix_pass(ka, va, kb, vb, hist, 0)
    _radix_pass(kb, vb, ka, va, hist, 8)
    _radix_pass(ka, va, kb, vb, hist, 16)
    _radix_pass(kb, vb, ka, va, hist, 24)

    _radix_pass(k2, va, kb, vb, hist, 0)
    _radix_pass(kb, vb, k2, va, hist, 8)
    _radix_pass(k2, va, kb, vb, hist, 16)
    _radix_pass(kb, vb, k2, va, hist, 24)

    pltpu.sync_copy(va.at[pl.ds(0, _NQ)], out_hbm.at[wid])


def _subkey_table():
    keys = jax.random.split(jax.random.key(42), _B)
    s1 = jax.vmap(jax.random.split)(keys)
    s2 = jax.vmap(jax.random.split)(s1[:, 0])
    d1 = jax.random.key_data(s1[:, 1])
    d2 = jax.random.key_data(s2[:, 1])
    return lax.bitcast_convert_type(
        jnp.concatenate([d1, d2], axis=1), jnp.int32).reshape(_B, 1, 4)


def kernel(points):
    del points
    sk = _subkey_table()
    bits = pl.pallas_call(
        _bits_kernel,
        grid=(_B,),
        in_specs=[pl.BlockSpec((1, 1, 4), lambda i: (i, 0, 0),
                               memory_space=pltpu.SMEM)],
        out_specs=pl.BlockSpec((1, 2, _SUB, _LANE), lambda i: (i, 0, 0, 0)),
        out_shape=jax.ShapeDtypeStruct((_B, 2, _SUB, _LANE), jnp.int32),
    )(sk)
    return _sc_sort(bits.reshape(_B, 2, _N))
